# Initial kernel scaffold; baseline (speedup 1.0000x reference)
import functools

import jax
import jax.numpy as jnp
from jax import lax
from jax.experimental import pallas as pl
from jax.experimental.pallas import tpu as pltpu
from jax.experimental.pallas import tpu_sc as plsc

NUM_EMB = 1000000
FEAT = 32
B_TOTAL = 16384 * 26
IDX_W = 128
N_ROWS = B_TOTAL // IDX_W
NC, NS = 2, 16
NW = NC * NS
K = N_ROWS // NW
G = 8
CH = K // G
CHUNK_ROWS = G * IDX_W


@functools.partial(
    pl.kernel,
    out_type=jax.ShapeDtypeStruct((B_TOTAL, FEAT), jnp.float32),
    mesh=plsc.VectorSubcoreMesh(core_axis_name="c", subcore_axis_name="s"),
    scratch_types=[
        pltpu.VMEM((K, IDX_W), jnp.int32),
        pltpu.VMEM((CHUNK_ROWS, FEAT), jnp.float32),
        pltpu.SemaphoreType.DMA,
    ],
)
def _embed_sc(idx_hbm, table_hbm, out_hbm, idx_v, rows_v, sem):
    wid = lax.axis_index("s") * NC + lax.axis_index("c")
    pltpu.sync_copy(idx_hbm.at[pl.ds(wid * K, K)], idx_v)
    out_base = wid * K * IDX_W

    def chunk_body(c, carry):
        copies = []
        for g in range(G):
            copies.append(
                pltpu.async_copy(
                    table_hbm.at[idx_v.at[c * G + g]],
                    rows_v.at[pl.ds(g * IDX_W, IDX_W)],
                    sem,
                )
            )
        for cp in copies:
            cp.wait()
        pltpu.sync_copy(
            rows_v, out_hbm.at[pl.ds(out_base + c * CHUNK_ROWS, CHUNK_ROWS)]
        )
        return carry

    lax.fori_loop(0, CH, chunk_body, 0)


def kernel(inputs, embedding):
    idx2d = inputs.reshape(N_ROWS, IDX_W)
    out = _embed_sc(idx2d, embedding)
    return out.reshape(inputs.shape[0], inputs.shape[1], FEAT)


# speedup vs baseline: 1.5592x; 1.5592x over previous
import functools

import jax
import jax.numpy as jnp
from jax import lax
from jax.experimental import pallas as pl
from jax.experimental.pallas import tpu as pltpu
from jax.experimental.pallas import tpu_sc as plsc

NUM_EMB = 1000000
FEAT = 32
B_TOTAL = 16384 * 26
IDX_W = 128
N_ROWS = B_TOTAL // IDX_W
NC, NS = 2, 16
NW = NC * NS
K = N_ROWS // NW
G = 8
CH = K // G
CHUNK_ROWS = G * IDX_W


@functools.partial(
    pl.kernel,
    out_type=jax.ShapeDtypeStruct((B_TOTAL, FEAT), jnp.float32),
    mesh=plsc.VectorSubcoreMesh(core_axis_name="c", subcore_axis_name="s"),
    scratch_types=[
        pltpu.VMEM((K, IDX_W), jnp.int32),
        pltpu.VMEM((CHUNK_ROWS, FEAT), jnp.float32),
        pltpu.SemaphoreType.DMA,
    ],
    compiler_params=pltpu.CompilerParams(use_tc_tiling_on_sc=False),
)
def _embed_sc(idx_hbm, table_hbm, out_hbm, idx_v, rows_v, sem):
    wid = lax.axis_index("s") * NC + lax.axis_index("c")
    pltpu.sync_copy(idx_hbm.at[pl.ds(wid * K, K)], idx_v)
    out_base = wid * K * IDX_W

    def chunk_body(c, carry):
        copies = []
        for g in range(G):
            copies.append(
                pltpu.async_copy(
                    table_hbm.at[idx_v.at[c * G + g]],
                    rows_v.at[pl.ds(g * IDX_W, IDX_W)],
                    sem,
                )
            )
        for cp in copies:
            cp.wait()
        pltpu.sync_copy(
            rows_v, out_hbm.at[pl.ds(out_base + c * CHUNK_ROWS, CHUNK_ROWS)]
        )
        return carry

    lax.fori_loop(0, CH, chunk_body, 0)


def kernel(inputs, embedding):
    idx2d = inputs.reshape(N_ROWS, IDX_W)
    out = _embed_sc(idx2d, embedding)
    return out.reshape(inputs.shape[0], inputs.shape[1], FEAT)


# speedup vs baseline: 1.5671x; 1.0050x over previous
import functools

import jax
import jax.numpy as jnp
from jax import lax
from jax.experimental import pallas as pl
from jax.experimental.pallas import tpu as pltpu
from jax.experimental.pallas import tpu_sc as plsc

NUM_EMB = 1000000
FEAT = 32
B_TOTAL = 16384 * 26
IDX_W = 128
N_ROWS = B_TOTAL // IDX_W
NC, NS = 2, 16
NW = NC * NS
K = N_ROWS // NW
G = 13
CH = K // G
CHUNK_ROWS = G * IDX_W


@functools.partial(
    pl.kernel,
    out_type=jax.ShapeDtypeStruct((B_TOTAL, FEAT), jnp.float32),
    mesh=plsc.VectorSubcoreMesh(core_axis_name="c", subcore_axis_name="s"),
    scratch_types=[
        pltpu.VMEM((K, IDX_W), jnp.int32),
        pltpu.VMEM((2, CHUNK_ROWS, FEAT), jnp.float32),
        pltpu.SemaphoreType.DMA,
        pltpu.SemaphoreType.DMA,
    ],
    compiler_params=pltpu.CompilerParams(use_tc_tiling_on_sc=False),
)
def _embed_sc(idx_hbm, table_hbm, out_hbm, idx_v, rows_v, sem_g, sem_o):
    wid = lax.axis_index("s") * NC + lax.axis_index("c")
    pltpu.sync_copy(idx_hbm.at[pl.ds(wid * K, K)], idx_v)
    out_base = wid * K * IDX_W

    def fire(c, buf):
        for g in range(G):
            pltpu.async_copy(
                table_hbm.at[idx_v.at[c * G + g]],
                rows_v.at[buf, pl.ds(g * IDX_W, IDX_W)],
                sem_g,
            )

    def drain_gathers():
        pltpu.make_async_copy(
            out_hbm.at[pl.ds(0, CHUNK_ROWS)], rows_v.at[0], sem_g
        ).wait()

    def start_out(c, buf):
        pltpu.async_copy(
            rows_v.at[buf],
            out_hbm.at[pl.ds(out_base + c * CHUNK_ROWS, CHUNK_ROWS)],
            sem_o,
        )

    def drain_out():
        pltpu.make_async_copy(
            rows_v.at[0], out_hbm.at[pl.ds(0, CHUNK_ROWS)], sem_o
        ).wait()

    fire(0, 0)
    drain_gathers()
    start_out(0, 0)
    fire(1, 1)

    def body(c, carry):
        buf = c % 2
        drain_gathers()
        start_out(c, buf)
        drain_out()
        fire(c + 1, 1 - buf)
        return carry

    lax.fori_loop(1, CH - 1, body, 0)

    drain_gathers()
    start_out(CH - 1, (CH - 1) % 2)
    drain_out()
    drain_out()


def kernel(inputs, embedding):
    idx2d = inputs.reshape(N_ROWS, IDX_W)
    out = _embed_sc(idx2d, embedding)
    return out.reshape(inputs.shape[0], inputs.shape[1], FEAT)
